# Initial kernel scaffold; baseline (speedup 1.0000x reference)
import jax
import jax.numpy as jnp
from jax.experimental import pallas as pl


def kernel(x, W_route, b_route, W_noise, b_noise, w1, b1, w2, b2, noise):
    raise NotImplementedError("write your pallas kernel here")


# speedup vs baseline: 1.3147x; 1.3147x over previous
import jax
import jax.numpy as jnp
from jax.experimental import pallas as pl
from jax.experimental.pallas import tpu as pltpu

D = 1024
E = 8
H = 4096
N = 2048
BH = 1024
BT = 1024
NH = H // BH
NT = N // BT


def _moe_dense_body(x_ref, wr_ref, br_ref, wn_ref, bn_ref, noise_ref,
                    w1_ref, b1_ref, w2_ref, b2_ref, out_ref, gates_ref):
    e = pl.program_id(1)
    h = pl.program_id(2)

    @pl.when(jnp.logical_and(e == 0, h == 0))
    def _router_and_init():
        xb = x_ref[...]
        logits = jnp.dot(xb, wr_ref[...]) + br_ref[...]
        nlog = jnp.dot(xb, wn_ref[...]) + bn_ref[...]
        sp = jnp.maximum(nlog, 0.0) + jnp.log1p(jnp.exp(-jnp.abs(nlog)))
        nl = logits + noise_ref[...] * sp
        lane = jax.lax.broadcasted_iota(jnp.int32, nl.shape, 1)
        m1 = jnp.max(nl, axis=1, keepdims=True)
        i1 = jnp.min(jnp.where(nl == m1, lane, E), axis=1, keepdims=True)
        mask1 = lane == i1
        nl2 = jnp.where(mask1, -jnp.inf, nl)
        m2 = jnp.max(nl2, axis=1, keepdims=True)
        i2 = jnp.min(jnp.where(nl2 == m2, lane, E), axis=1, keepdims=True)
        mask2 = lane == i2
        e2 = jnp.exp(m2 - m1)
        denom = 1.0 + e2
        gates_ref[...] = (jnp.where(mask1, 1.0 / denom, 0.0)
                          + jnp.where(mask2, e2 / denom, 0.0))
        out_ref[...] = jnp.zeros_like(out_ref)

    lane = jax.lax.broadcasted_iota(jnp.int32, (BT, E), 1)
    col = jnp.sum(jnp.where(lane == e, gates_ref[...], 0.0), axis=1,
                  keepdims=True)

    @pl.when(h == 0)
    def _add_b2():
        out_ref[...] += col * b2_ref[0]

    xb = x_ref[...].astype(jnp.bfloat16)
    hp = jnp.dot(xb, w1_ref[0].astype(jnp.bfloat16),
                 preferred_element_type=jnp.float32) + b1_ref[0]
    hp = jnp.maximum(hp, 0.0).astype(jnp.bfloat16)
    yp = jnp.dot(hp, w2_ref[0].astype(jnp.bfloat16),
                 preferred_element_type=jnp.float32)
    out_ref[...] += col * yp


def kernel(x, W_route, b_route, W_noise, b_noise, w1, b1, w2, b2, noise):
    grid = (NT, E, NH)
    out = pl.pallas_call(
        _moe_dense_body,
        grid=grid,
        in_specs=[
            pl.BlockSpec((BT, D), lambda t, e, h: (t, 0)),
            pl.BlockSpec((D, E), lambda t, e, h: (0, 0)),
            pl.BlockSpec((1, E), lambda t, e, h: (0, 0)),
            pl.BlockSpec((D, E), lambda t, e, h: (0, 0)),
            pl.BlockSpec((1, E), lambda t, e, h: (0, 0)),
            pl.BlockSpec((BT, E), lambda t, e, h: (t, 0)),
            pl.BlockSpec((1, D, BH), lambda t, e, h: (e, 0, h)),
            pl.BlockSpec((1, 1, BH), lambda t, e, h: (e, 0, h)),
            pl.BlockSpec((1, BH, D), lambda t, e, h: (e, h, 0)),
            pl.BlockSpec((1, 1, D), lambda t, e, h: (e, 0, 0)),
        ],
        out_specs=pl.BlockSpec((BT, D), lambda t, e, h: (t, 0)),
        out_shape=jax.ShapeDtypeStruct((N, D), jnp.float32),
        scratch_shapes=[pltpu.VMEM((BT, E), jnp.float32)],
        compiler_params=pltpu.CompilerParams(
            dimension_semantics=("arbitrary", "arbitrary", "arbitrary"),
        ),
    )(x, W_route, b_route.reshape(1, E), W_noise, b_noise.reshape(1, E),
      noise, w1, b1.reshape(E, 1, H), w2, b2.reshape(E, 1, D))
    return out
